# baseline (speedup 1.0000x reference)
import jax
import jax.numpy as jnp
from jax.experimental import pallas as pl
from jax.sharding import PartitionSpec as P


def _proj_kernel(x_ref, wt_ref, y_ref):
    y_ref[...] = jnp.dot(x_ref[...], wt_ref[...],
                         preferred_element_type=jnp.float32).astype(jnp.bfloat16)


def _main_kernel(a_ref, y_ref, b_ref, o_ref):
    acc = jnp.dot(a_ref[...].astype(jnp.bfloat16), y_ref[...],
                  preferred_element_type=jnp.float32)
    o_ref[...] = jnp.maximum(acc + b_ref[...], 0.0)


def _shard_impl(x, a_shard, wt, b2d):
    n, d_in = x.shape
    d_out = wt.shape[1]
    m = a_shard.shape[0]

    y = pl.pallas_call(
        _proj_kernel,
        out_shape=jax.ShapeDtypeStruct((n, d_out), jnp.bfloat16),
    )(x, wt)

    bm = 200
    return pl.pallas_call(
        _main_kernel,
        grid=(m // bm,),
        in_specs=[
            pl.BlockSpec((bm, n), lambda i: (i, 0)),
            pl.BlockSpec((n, d_out), lambda i: (0, 0)),
            pl.BlockSpec((1, d_out), lambda i: (0, 0)),
        ],
        out_specs=pl.BlockSpec((bm, d_out), lambda i: (i, 0)),
        out_shape=jax.ShapeDtypeStruct((m, d_out), jnp.float32),
    )(a_shard, y, b2d)


def kernel(node_features, adjacency_matrix, W, b):
    n = adjacency_matrix.shape[0]
    d_out = W.shape[0]
    wt = W.T
    b2d = b.reshape(1, d_out)

    ndev = jax.device_count()
    if n % ndev != 0 or (n // ndev) % 200 != 0:
        ndev = 1

    if ndev == 1:
        return _shard_impl(node_features, adjacency_matrix, wt, b2d)

    mesh = jax.make_mesh((ndev,), ("x",))
    ns = lambda spec: jax.NamedSharding(mesh, spec)
    node_features = jax.reshard(node_features, ns(P()))
    adjacency_matrix = jax.reshard(adjacency_matrix, ns(P("x", None)))
    wt = jax.reshard(wt, ns(P()))
    b2d = jax.reshard(b2d, ns(P()))
    return jax.shard_map(
        _shard_impl,
        mesh=mesh,
        in_specs=(P(), P("x", None), P(), P()),
        out_specs=P("x", None),
        check_vma=False,
    )(node_features, adjacency_matrix, wt, b2d)


# speedup vs baseline: 5.4921x; 5.4921x over previous
import jax
import jax.numpy as jnp
from jax.experimental import pallas as pl
from jax.experimental.pallas import tpu as pltpu


def _main_kernel(a1_ref, a2_ref, x_ref, wt_ref, b_ref, o1_ref, o2_ref, y_ref):
    @pl.when(pl.program_id(0) == 0)
    def _():
        y_ref[...] = jnp.dot(x_ref[...], wt_ref[...],
                             preferred_element_type=jnp.float32
                             ).astype(jnp.bfloat16)

    y = y_ref[...]
    b_vec = b_ref[...]
    acc1 = jnp.dot(a1_ref[0].astype(jnp.bfloat16), y,
                   preferred_element_type=jnp.float32)
    o1_ref[...] = jnp.maximum(acc1 + b_vec, 0.0)
    acc2 = jnp.dot(a2_ref[0].astype(jnp.bfloat16), y,
                   preferred_element_type=jnp.float32)
    o2_ref[...] = jnp.maximum(acc2 + b_vec, 0.0)


def kernel(node_features, adjacency_matrix, W, b):
    n, d_in = node_features.shape
    d_out = W.shape[0]
    wt = W.T
    b2d = b.reshape(1, d_out)
    h = n // 2
    a3 = adjacency_matrix.reshape(2, h, n)

    bm = 200
    o1, o2 = pl.pallas_call(
        _main_kernel,
        grid=(h // bm,),
        in_specs=[
            pl.BlockSpec((1, bm, n), lambda i: (0, i, 0)),
            pl.BlockSpec((1, bm, n), lambda i: (1, i, 0)),
            pl.BlockSpec((n, d_in), lambda i: (0, 0)),
            pl.BlockSpec((d_in, d_out), lambda i: (0, 0)),
            pl.BlockSpec((1, d_out), lambda i: (0, 0)),
        ],
        out_specs=[
            pl.BlockSpec((bm, d_out), lambda i: (i, 0)),
            pl.BlockSpec((bm, d_out), lambda i: (i, 0)),
        ],
        out_shape=[
            jax.ShapeDtypeStruct((h, d_out), jnp.float32),
            jax.ShapeDtypeStruct((h, d_out), jnp.float32),
        ],
        scratch_shapes=[pltpu.VMEM((n, d_out), jnp.bfloat16)],
    )(a3, a3, node_features, wt, b2d)
    return jnp.concatenate([o1, o2], axis=0)


# speedup vs baseline: 5.5713x; 1.0144x over previous
import jax
import jax.numpy as jnp
from jax.experimental import pallas as pl
from jax.experimental.pallas import tpu as pltpu

_BM = 200
_DEPTH = 4


def _main_kernel(nb, a_hbm, x_hbm, wt_ref, b_ref, o_ref,
                 a_buf, x_buf, y_ref, a_sems, x_sem):
    i = pl.program_id(0)
    bm = a_buf.shape[1]

    @pl.when(i == 0)
    def _():
        for j in range(_DEPTH - 1):
            pltpu.make_async_copy(
                a_hbm.at[pl.ds(j * bm, bm), :], a_buf.at[j], a_sems.at[j]
            ).start()
        xcopy = pltpu.make_async_copy(x_hbm, x_buf, x_sem)
        xcopy.start()
        xcopy.wait()
        y_ref[...] = jnp.dot(x_buf[...], wt_ref[...],
                             preferred_element_type=jnp.float32
                             ).astype(jnp.bfloat16)

    slot = jax.lax.rem(i, _DEPTH)
    pltpu.make_async_copy(
        a_hbm.at[pl.ds(i * bm, bm), :], a_buf.at[slot], a_sems.at[slot]
    ).wait()
    acc = jnp.dot(a_buf[slot].astype(jnp.bfloat16), y_ref[...],
                  preferred_element_type=jnp.float32)
    o_ref[...] = jnp.maximum(acc + b_ref[...], 0.0)

    nxt = i + _DEPTH - 1

    @pl.when(nxt < nb)
    def _():
        nslot = jax.lax.rem(nxt, _DEPTH)
        pltpu.make_async_copy(
            a_hbm.at[pl.ds(nxt * bm, bm), :], a_buf.at[nslot],
            a_sems.at[nslot]
        ).start()


def kernel(node_features, adjacency_matrix, W, b):
    n, d_in = node_features.shape
    d_out = W.shape[0]
    wt = W.T
    b2d = b.reshape(1, d_out)
    nb = n // _BM

    import functools
    return pl.pallas_call(
        functools.partial(_main_kernel, nb),
        grid=(nb,),
        in_specs=[
            pl.BlockSpec(memory_space=pl.ANY),
            pl.BlockSpec(memory_space=pl.ANY),
            pl.BlockSpec((d_in, d_out), lambda i: (0, 0)),
            pl.BlockSpec((1, d_out), lambda i: (0, 0)),
        ],
        out_specs=pl.BlockSpec((_BM, d_out), lambda i: (i, 0)),
        out_shape=jax.ShapeDtypeStruct((n, d_out), jnp.float32),
        scratch_shapes=[
            pltpu.VMEM((_DEPTH, _BM, n), jnp.float32),
            pltpu.VMEM((n, d_in), jnp.float32),
            pltpu.VMEM((n, d_out), jnp.bfloat16),
            pltpu.SemaphoreType.DMA((_DEPTH,)),
            pltpu.SemaphoreType.DMA,
        ],
    )(adjacency_matrix, node_features, wt, b2d)
